# baseline (speedup 1.0000x reference)
import functools

import jax
import jax.numpy as jnp
from jax.experimental import pallas as pl
from jax.experimental.pallas import tpu as pltpu


def _sigm(z):
    return 0.5 * jnp.tanh(0.5 * z) + 0.5


def _step_mxu(xs_ref, wx_ref, wh_ref, b_ref, hs_ref, k, c01, h01):
    x = xs_ref[k].astype(jnp.bfloat16)
    xk = (x[:, 0:256], x[:, 256:512])

    for j in range(4):
        for kk in range(2):
            pltpu.matmul_push_rhs(
                wx_ref[kk * 256:(kk + 1) * 256, j * 256:(j + 1) * 256],
                staging_register=kk, mxu_index=0)
            pltpu.matmul_push_rhs(
                wx_ref[kk * 256:(kk + 1) * 256, (4 + j) * 256:(5 + j) * 256],
                staging_register=kk, mxu_index=1)
            pltpu.matmul_acc_lhs(j * 64, xk[kk], mxu_index=0,
                                 load_staged_rhs=kk)
            pltpu.matmul_acc_lhs(j * 64, xk[kk], mxu_index=1,
                                 load_staged_rhs=kk)

    g = [None] * 8
    for j in range(4):
        for kk in range(2):
            pltpu.matmul_push_rhs(
                wh_ref[kk * 256:(kk + 1) * 256, j * 256:(j + 1) * 256],
                staging_register=kk, mxu_index=0)
            pltpu.matmul_push_rhs(
                wh_ref[kk * 256:(kk + 1) * 256, (4 + j) * 256:(5 + j) * 256],
                staging_register=kk, mxu_index=1)
            pltpu.matmul_acc_lhs(j * 64, h01[kk], mxu_index=0,
                                 load_staged_rhs=kk)
            pltpu.matmul_acc_lhs(j * 64, h01[kk], mxu_index=1,
                                 load_staged_rhs=kk)
        g[j] = (pltpu.matmul_pop(j * 64, (256, 256), jnp.float32, mxu_index=0)
                + b_ref[:, j * 256:(j + 1) * 256])
        g[4 + j] = (pltpu.matmul_pop(j * 64, (256, 256), jnp.float32,
                                     mxu_index=1)
                    + b_ref[:, (4 + j) * 256:(5 + j) * 256])

    c_new, h_new, h_bf = [None, None], [None, None], [None, None]
    for cb in range(2):
        i_g = _sigm(g[0 + cb])
        f_g = _sigm(g[2 + cb])
        g_c = jnp.tanh(g[4 + cb])
        o_g = _sigm(g[6 + cb])
        c_new[cb] = f_g * c01[cb] + i_g * g_c
        h_new[cb] = o_g * jnp.tanh(c_new[cb])
        h_bf[cb] = h_new[cb].astype(jnp.bfloat16)
        hs_ref[k, :, cb * 256:(cb + 1) * 256] = h_new[cb]
    return c_new, h_bf


def _seq_body_mxu(xs_ref, c0_ref, h0_ref, wx_ref, wh_ref, b_ref,
                  hs_ref, c_fin_ref, c_s, h_s, *, tt):
    blk = pl.program_id(0)

    @pl.when(blk == 0)
    def _():
        c_s[...] = c0_ref[...]
        h_s[...] = h0_ref[...]

    c01 = [c_s[:, 0:256], c_s[:, 256:512]]
    h01 = [h_s[:, 0:256].astype(jnp.bfloat16),
           h_s[:, 256:512].astype(jnp.bfloat16)]
    for k in range(tt):
        c01, h01 = _step_mxu(xs_ref, wx_ref, wh_ref, b_ref, hs_ref,
                             k, c01, h01)

    for cb in range(2):
        c_s[:, cb * 256:(cb + 1) * 256] = c01[cb]
        h_s[:, cb * 256:(cb + 1) * 256] = h01[cb].astype(jnp.float32)

    @pl.when(blk == pl.num_programs(0) - 1)
    def _():
        for cb in range(2):
            c_fin_ref[:, cb * 256:(cb + 1) * 256] = c01[cb]


def _seq_body_gen(xs_ref, c0_ref, h0_ref, wx_ref, wh_ref, b_ref,
                  hs_ref, c_fin_ref, c_s, h_s, *, hp, tt):
    blk = pl.program_id(0)

    @pl.when(blk == 0)
    def _():
        c_s[...] = c0_ref[...]
        h_s[...] = h0_ref[...]

    bt = c_s.shape[0]
    nh = 2 if bt % 256 == 0 else 1
    bh = bt // nh

    cs = [c_s[j * bh:(j + 1) * bh, :] for j in range(nh)]
    hs = [h_s[j * bh:(j + 1) * bh, :] for j in range(nh)]
    for k in range(tt):
        for j in range(nh):
            x = xs_ref[k, j * bh:(j + 1) * bh, :].astype(jnp.bfloat16)
            gates = (jnp.dot(x, wx_ref[...],
                             preferred_element_type=jnp.float32)
                     + jnp.dot(hs[j].astype(jnp.bfloat16), wh_ref[...],
                               preferred_element_type=jnp.float32)
                     + b_ref[...])
            i_g = _sigm(gates[:, 0 * hp:1 * hp])
            f_g = _sigm(gates[:, 1 * hp:2 * hp])
            g_c = jnp.tanh(gates[:, 2 * hp:3 * hp])
            o_g = _sigm(gates[:, 3 * hp:4 * hp])
            cs[j] = f_g * cs[j] + i_g * g_c
            hs[j] = o_g * jnp.tanh(cs[j])
            hs_ref[k, j * bh:(j + 1) * bh, :] = hs[j]

    for j in range(nh):
        c_s[j * bh:(j + 1) * bh, :] = cs[j]
        h_s[j * bh:(j + 1) * bh, :] = hs[j]

    @pl.when(blk == pl.num_programs(0) - 1)
    def _():
        for j in range(nh):
            c_fin_ref[j * bh:(j + 1) * bh, :] = cs[j]


def kernel(xs, c0, h0, wx, wh, b):
    T, B, D = xs.shape
    H = h0.shape[1]
    Hp4 = wx.shape[1]
    Hp = Hp4 // 4

    if Hp != H:
        c0 = jnp.pad(c0, ((0, 0), (0, Hp - H)))
        h0 = jnp.pad(h0, ((0, 0), (0, Hp - H)))

    wx_b = wx.astype(jnp.bfloat16)
    wh_b = wh.astype(jnp.bfloat16)
    b_f = b.astype(jnp.float32)

    tt = 8
    while T % tt:
        tt //= 2

    if B == 256 and D == 512 and Hp == 512:
        body = functools.partial(_seq_body_mxu, tt=tt)
    else:
        body = functools.partial(_seq_body_gen, hp=Hp, tt=tt)

    hs, c_fin = pl.pallas_call(
        body,
        out_shape=(
            jax.ShapeDtypeStruct((T, B, Hp), h0.dtype),
            jax.ShapeDtypeStruct((B, Hp), c0.dtype),
        ),
        grid=(T // tt,),
        in_specs=[
            pl.BlockSpec((tt, B, D), lambda i: (i, 0, 0)),
            pl.BlockSpec((B, Hp), lambda i: (0, 0)),
            pl.BlockSpec((B, Hp), lambda i: (0, 0)),
            pl.BlockSpec((D, Hp4), lambda i: (0, 0)),
            pl.BlockSpec((Hp, Hp4), lambda i: (0, 0)),
            pl.BlockSpec((1, Hp4), lambda i: (0, 0)),
        ],
        out_specs=(
            pl.BlockSpec((tt, B, Hp), lambda i: (i, 0, 0)),
            pl.BlockSpec((B, Hp), lambda i: (0, 0)),
        ),
        scratch_shapes=[
            pltpu.VMEM((B, Hp), jnp.float32),
            pltpu.VMEM((B, Hp), jnp.float32),
        ],
        compiler_params=pltpu.CompilerParams(
            dimension_semantics=("arbitrary",),
            vmem_limit_bytes=56 * 1024 * 1024,
        ),
    )(xs, c0, h0, wx_b, wh_b, b_f)

    h_fin = hs[-1]
    if Hp != H:
        hs, c_fin, h_fin = hs[:, :, :H], c_fin[:, :H], h_fin[:, :H]
    return hs, c_fin, h_fin


# speedup vs baseline: 1.1603x; 1.1603x over previous
import functools

import jax
import jax.numpy as jnp
from jax.experimental import pallas as pl
from jax.experimental.pallas import tpu as pltpu


def _round_up(x, m):
    return (x + m - 1) // m * m


def _sigm(z):
    return jax.nn.sigmoid(z)


def _seq_body(xs_ref, c0_ref, h0_ref, wx_ref, wh_ref, b_ref,
              hs_ref, c_fin_ref, c_s, h_s, *, hp, tt):
    blk = pl.program_id(0)

    @pl.when(blk == 0)
    def _():
        c_s[...] = c0_ref[...]
        h_s[...] = h0_ref[...]

    bt = c_s.shape[0]
    nh = 2 if bt % 256 == 0 else 1
    bh = bt // nh

    cs = [c_s[j * bh:(j + 1) * bh, :] for j in range(nh)]
    hs = [h_s[j * bh:(j + 1) * bh, :] for j in range(nh)]
    for k in range(tt):
        for j in range(nh):
            x = xs_ref[k, j * bh:(j + 1) * bh, :].astype(jnp.bfloat16)
            gates = (jnp.dot(x, wx_ref[...],
                             preferred_element_type=jnp.float32)
                     + jnp.dot(hs[j].astype(jnp.bfloat16), wh_ref[...],
                               preferred_element_type=jnp.float32)
                     + b_ref[...])
            i_g = _sigm(gates[:, 0 * hp:1 * hp])
            f_g = _sigm(gates[:, 1 * hp:2 * hp])
            g_c = jnp.tanh(gates[:, 2 * hp:3 * hp])
            o_g = _sigm(gates[:, 3 * hp:4 * hp])
            cs[j] = f_g * cs[j] + i_g * g_c
            hs[j] = o_g * jnp.tanh(cs[j])
            hs_ref[k, j * bh:(j + 1) * bh, :] = hs[j]

    for j in range(nh):
        c_s[j * bh:(j + 1) * bh, :] = cs[j]
        h_s[j * bh:(j + 1) * bh, :] = hs[j]

    @pl.when(blk == pl.num_programs(0) - 1)
    def _():
        for j in range(nh):
            c_fin_ref[j * bh:(j + 1) * bh, :] = cs[j]


def kernel(xs, c0, h0, wx, wh, b):
    T, B, D = xs.shape
    H = h0.shape[1]
    Hp4 = wx.shape[1]
    Hp = Hp4 // 4

    if Hp != H:
        c0 = jnp.pad(c0, ((0, 0), (0, Hp - H)))
        h0 = jnp.pad(h0, ((0, 0), (0, Hp - H)))

    wx_b = wx.astype(jnp.bfloat16)
    wh_b = wh.astype(jnp.bfloat16)
    b_f = b.astype(jnp.float32)

    tt = 8
    while T % tt:
        tt //= 2

    body = functools.partial(_seq_body, hp=Hp, tt=tt)

    hs, c_fin = pl.pallas_call(
        body,
        out_shape=(
            jax.ShapeDtypeStruct((T, B, Hp), h0.dtype),
            jax.ShapeDtypeStruct((B, Hp), c0.dtype),
        ),
        grid=(T // tt,),
        in_specs=[
            pl.BlockSpec((tt, B, D), lambda i: (i, 0, 0)),
            pl.BlockSpec((B, Hp), lambda i: (0, 0)),
            pl.BlockSpec((B, Hp), lambda i: (0, 0)),
            pl.BlockSpec((D, Hp4), lambda i: (0, 0)),
            pl.BlockSpec((Hp, Hp4), lambda i: (0, 0)),
            pl.BlockSpec((1, Hp4), lambda i: (0, 0)),
        ],
        out_specs=(
            pl.BlockSpec((tt, B, Hp), lambda i: (i, 0, 0)),
            pl.BlockSpec((B, Hp), lambda i: (0, 0)),
        ),
        scratch_shapes=[
            pltpu.VMEM((B, Hp), jnp.float32),
            pltpu.VMEM((B, Hp), jnp.float32),
        ],
        compiler_params=pltpu.CompilerParams(
            dimension_semantics=("arbitrary",),
            vmem_limit_bytes=56 * 1024 * 1024,
        ),
    )(xs, c0, h0, wx_b, wh_b, b_f)

    h_fin = hs[-1]
    if Hp != H:
        hs, c_fin, h_fin = hs[:, :, :H], c_fin[:, :H], h_fin[:, :H]
    return hs, c_fin, h_fin
